# baseline (speedup 1.0000x reference)
import functools

import jax
import jax.numpy as jnp
import numpy as np
from jax import lax
from jax.experimental import pallas as pl
from jax.experimental.pallas import tpu as pltpu
from jax.experimental.pallas import tpu_sc as plsc

N = 10000
E = 320000
D = 128
H = 8
F = 64
HF = 512
P = 2
NC, NS, L = 2, 16, 16
NW = NC * NS

BN = 1000
NB = N // BN

ECH = 128
KSUB = 5
BIG = ECH * KSUB
NBC = E // BIG
ER2 = E // ECH


def _proj_body(feat_ref, w_ref, alr_ref, h4_ref, elr_ref):
    f = feat_ref[...]
    w = w_ref[0]
    for hp in range(4):
        h4_ref[0, hp] = jnp.dot(f, w[:, hp * 128:(hp + 1) * 128],
                                preferred_element_type=jnp.float32)
    wlr = jnp.dot(w, alr_ref[0], preferred_element_type=jnp.float32)
    elr_ref[0] = jnp.dot(f, wlr, preferred_element_type=jnp.float32)


def _stage1(feat, W, ALR):
    return pl.pallas_call(
        _proj_body,
        grid=(P, NB),
        in_specs=[
            pl.BlockSpec((BN, D), lambda p, nb: (nb, 0)),
            pl.BlockSpec((1, D, HF), lambda p, nb: (p, 0, 0)),
            pl.BlockSpec((1, HF, 128), lambda p, nb: (p, 0, 0)),
        ],
        out_specs=[
            pl.BlockSpec((1, 4, BN, 128), lambda p, nb: (p, 0, nb, 0)),
            pl.BlockSpec((1, BN, 128), lambda p, nb: (p, nb, 0)),
        ],
        out_shape=[
            jax.ShapeDtypeStruct((P, 4, N, 128), jnp.float32),
            jax.ShapeDtypeStruct((P, N, 128), jnp.float32),
        ],
    )(feat, W, ALR)


_MESH = plsc.VectorSubcoreMesh(core_axis_name="c", subcore_axis_name="s",
                               num_cores=NC, num_subcores=NS)
_MESH1 = plsc.VectorSubcoreMesh(core_axis_name="c", subcore_axis_name="s",
                                num_cores=1, num_subcores=NS)


def _row_split(s):
    start = jnp.where(s < 2, s * 632, 1264 + (s - 2) * 624)
    return pl.multiple_of(start, 8)


def _load_idx(arr, base, buf, sem):
    cps = [pltpu.async_copy(arr.at[pl.ds(base + r * ECH, ECH)],
                            buf.at[r], sem) for r in range(KSUB)]
    for cp in cps:
        cp.wait()


def _stats_body(elr0_h, elr1_h, src0, dst0, src1, dst1, ee_hbm,
                srcv, dstv, gs, wl, wr, semg, semi):
    s = lax.axis_index("s")
    elrs = (elr0_h, elr1_h)
    srcs = (src0, src1)
    dsts = (dst0, dst1)

    nbc = jnp.where(s < (NBC % NS), NBC // NS + 1, NBC // NS)
    for p in range(P):

        def _chunk(k, _):
            bc = s + k * NS
            base = pl.multiple_of(bc * BIG, 8)
            _load_idx(srcs[p], base, srcv, semi)
            _load_idx(dsts[p], base, dstv, semi)

            for r in range(KSUB):
                pltpu.async_copy(elrs[p].at[srcv.at[r]], wl, semg).wait()
                pltpu.async_copy(elrs[p].at[dstv.at[r]], wr, semg).wait()

                def _cmp(i, _):
                    x = wl[i, pl.ds(0, 16)] + wr[i, pl.ds(8, 16)]
                    gs[r * ECH + i, :] = jnp.exp(jnp.maximum(x, 0.2 * x))
                    return 0
                lax.fori_loop(0, ECH, _cmp, 0)

            pltpu.sync_copy(gs, ee_hbm.at[p, pl.ds(base, BIG)])
            return 0
        lax.fori_loop(0, nbc, _chunk, 0)


def _stats(elr0, elr1, src0, dst0, src1, dst1):
    kk = pl.kernel(
        _stats_body,
        out_type=jax.ShapeDtypeStruct((P, E, 16), jnp.float32),
        mesh=_MESH1,
        scratch_types=[
            pltpu.VMEM((KSUB, ECH), jnp.int32),
            pltpu.VMEM((KSUB, ECH), jnp.int32),
            pltpu.VMEM((BIG, 16), jnp.float32),
            pltpu.VMEM((ECH, 128), jnp.float32),
            pltpu.VMEM((ECH, 128), jnp.float32),
            pltpu.SemaphoreType.DMA,
            pltpu.SemaphoreType.DMA,
        ],
        compiler_params=pltpu.CompilerParams(use_tc_tiling_on_sc=False),
    )
    return kk(elr0, elr1, src0, dst0, src1, dst1)


def _agg_body(h5f, eef, src2d, dst2d, raw_hbm,
              srcv, dstv, eeb, hb, hb2, zb, acc,
              semg, semg2, sems, sems2, semi):
    s = lax.axis_index("s")

    tstart = pl.multiple_of(jnp.where(s == 0, 0, 320 + (s - 1) * 312), 8)

    def _z(i, _):
        for v in range(8):
            zb[i, pl.ds(v * L, L)] = jnp.zeros((L,), jnp.float32)
        return 0
    lax.fori_loop(0, 104, _z, 0)

    nbc = jnp.where(s < (NBC % NS), NBC // NS + 1, NBC // NS)

    for hp in range(5):
        col0 = 2 * hp

        def _round(p, _):
            hbase = (p * 5 + hp) * N
            for m in range(2):
                mbase = m * 5000

                for q in range(3):
                    off = pl.multiple_of(q * 104, 8)
                    pltpu.sync_copy(zb, acc.at[pl.ds(tstart + off, 104)])

                @pl.when(s == 0)
                def _():
                    pltpu.sync_copy(zb.at[pl.ds(0, 8)],
                                    acc.at[pl.ds(312, 8)])
                plsc.subcore_barrier()

                def _chunk(k, _):
                    bc = s + k * NS
                    erow = p * (E // ECH) + bc * KSUB
                    ebase = pl.multiple_of(p * E + bc * BIG, 8)
                    c1 = pltpu.async_copy(src2d.at[pl.ds(erow, KSUB)],
                                          srcv, semi)
                    c2 = pltpu.async_copy(dst2d.at[pl.ds(erow, KSUB)],
                                          dstv, semi)
                    c3 = pltpu.async_copy(eef.at[pl.ds(ebase, BIG)],
                                          eeb, semi)
                    c1.wait()
                    c2.wait()
                    c3.wait()
                    for r in range(KSUB):
                        for v in range(ECH // L):
                            sl = pl.ds(v * L, L)
                            srcv[r, sl] = srcv[r, sl] + hbase
                            t = dstv[r, sl] - mbase
                            ok = (t >= 0) & (t < 5000)
                            dstv[r, sl] = jnp.where(ok, t, 5000)

                    hbs = (hb, hb2)
                    sgs = (semg, semg2)
                    sss = (sems, sems2)
                    if hp < 4:
                        pltpu.async_copy(h5f.at[srcv.at[0]], hb, sgs[0])
                    for r in range(KSUB):
                        b = hbs[r % 2]
                        if hp < 4:
                            if r + 1 < KSUB:
                                ob = hbs[(r + 1) % 2]
                                if r >= 1:
                                    pltpu.make_async_copy(
                                        ob, acc.at[dstv.at[r - 1]],
                                        sss[(r + 1) % 2]).wait()
                                pltpu.async_copy(h5f.at[srcv.at[r + 1]],
                                                 ob, sgs[(r + 1) % 2])
                            pltpu.make_async_copy(h5f.at[srcv.at[r]],
                                                  b, sgs[r % 2]).wait()

                            @plsc.parallel_loop(0, ECH, unroll=4)
                            def _scale(i):
                                ev = eeb[r * ECH + i, :]
                                s0 = ev[col0]
                                s1 = ev[col0 + 1]
                                for v in range(4):
                                    b[i, pl.ds(v * L, L)] = (
                                        b[i, pl.ds(v * L, L)] * s0)
                                for v in range(4, 8):
                                    b[i, pl.ds(v * L, L)] = (
                                        b[i, pl.ds(v * L, L)] * s1)
                        else:
                            @plsc.parallel_loop(0, ECH, unroll=1)
                            def _scale(i):
                                ev = eeb[r * ECH + i, :]
                                for v in range(8):
                                    b[i, pl.ds(v * L, L)] = (
                                        jnp.ones((L,), jnp.float32) * ev[v])

                        cp = pltpu.async_copy(b, acc.at[dstv.at[r]],
                                              sss[r % 2], add=True)
                        if hp >= 4 or r >= KSUB - 2:
                            cp.wait()
                    return 0
                lax.fori_loop(0, nbc, _chunk, 0)

                plsc.subcore_barrier()
                rbase = pl.multiple_of(hbase + mbase + tstart, 8)
                for q in range(3):
                    off = pl.multiple_of(q * 104, 8)
                    pltpu.sync_copy(acc.at[pl.ds(tstart + off, 104)],
                                    raw_hbm.at[pl.ds(rbase + off, 104)])

                @pl.when(s == 0)
                def _():
                    pltpu.sync_copy(acc.at[pl.ds(312, 8)],
                                    raw_hbm.at[pl.ds(rbase + 312, 8)])
            return 0
        lax.fori_loop(0, P, _round, 0)


def _agg(h5f, eef, src2d, dst2d):
    kk = pl.kernel(
        _agg_body,
        out_type=jax.ShapeDtypeStruct((P * 5 * N, 128), jnp.float32),
        mesh=_MESH1,
        scratch_types=[
            pltpu.VMEM((KSUB, ECH), jnp.int32),
            pltpu.VMEM((KSUB, ECH), jnp.int32),
            pltpu.VMEM((BIG, 16), jnp.float32),
            pltpu.VMEM((ECH, 128), jnp.float32),
            pltpu.VMEM((ECH, 128), jnp.float32),
            pltpu.VMEM((104, 128), jnp.float32),
            pltpu.VMEM_SHARED((5008, 128), jnp.float32),
            pltpu.SemaphoreType.DMA,
            pltpu.SemaphoreType.DMA,
            pltpu.SemaphoreType.DMA,
            pltpu.SemaphoreType.DMA,
            pltpu.SemaphoreType.DMA,
        ],
        compiler_params=pltpu.CompilerParams(use_tc_tiling_on_sc=False),
    )
    return kk(h5f, eef, src2d, dst2d)


def _c1_body(raw_ref, sw1_ref, sb1_ref, sw2_ref, sel_ref,
             z_ref, wsum_ref):
    p = pl.program_id(0)
    nb = pl.program_id(1)
    d = raw_ref[0, 4]
    wacc = jnp.zeros((BN, 128), jnp.float32)
    for hp in range(4):
        den = jnp.dot(d, sel_ref[hp],
                      preferred_element_type=jnp.float32) + 1e-9
        zhp = raw_ref[0, hp] / den
        zhp = jnp.where(zhp > 0, zhp, jnp.exp(jnp.minimum(zhp, 0.0)) - 1.0)
        z_ref[0, :, hp * 128:(hp + 1) * 128] = zhp
        wacc = wacc + jnp.dot(zhp, sw1_ref[hp * 128:(hp + 1) * 128, :],
                              preferred_element_type=jnp.float32)
    w = jnp.dot(jnp.tanh(wacc + sb1_ref[...]), sw2_ref[...],
                preferred_element_type=jnp.float32)
    sc = jnp.sum(w)

    @pl.when(nb == 0)
    def _():
        wsum_ref[p, 0] = 0.0

    wsum_ref[p, 0] = wsum_ref[p, 0] + sc


def _c1(raw4, sW1, sb1r, sW2, SEL):
    return pl.pallas_call(
        _c1_body,
        grid=(P, NB),
        in_specs=[
            pl.BlockSpec((1, 5, BN, 128), lambda p, nb: (p, 0, nb, 0)),
            pl.BlockSpec((HF, 128), lambda p, nb: (0, 0)),
            pl.BlockSpec((1, 128), lambda p, nb: (0, 0)),
            pl.BlockSpec((128, 1), lambda p, nb: (0, 0)),
            pl.BlockSpec((4, 128, 128), lambda p, nb: (0, 0, 0)),
        ],
        out_specs=[
            pl.BlockSpec((1, BN, HF), lambda p, nb: (p, nb, 0)),
            pl.BlockSpec((P, 1), lambda p, nb: (0, 0),
                         memory_space=pltpu.SMEM),
        ],
        out_shape=[
            jax.ShapeDtypeStruct((P, N, HF), jnp.float32),
            jax.ShapeDtypeStruct((P, 1), jnp.float32),
        ],
    )(raw4, sW1, sb1r, sW2, SEL)


def _c2_body(z_ref, wsum_ref, out_ref):
    w0 = wsum_ref[0, 0] / N
    w1 = wsum_ref[1, 0] / N
    m = jnp.maximum(w0, w1)
    b0 = jnp.exp(w0 - m)
    b1 = jnp.exp(w1 - m)
    t = b0 + b1
    out_ref[...] = (b0 / t) * z_ref[0] + (b1 / t) * z_ref[1]


def _c2(z, wsum):
    return pl.pallas_call(
        _c2_body,
        grid=(NB,),
        in_specs=[
            pl.BlockSpec((P, BN, HF), lambda nb: (0, nb, 0)),
            pl.BlockSpec((P, 1), lambda nb: (0, 0), memory_space=pltpu.SMEM),
        ],
        out_specs=pl.BlockSpec((BN, HF), lambda nb: (nb, 0)),
        out_shape=jax.ShapeDtypeStruct((N, HF), jnp.float32),
    )(z, wsum)


@jax.jit
def _run(feat, edge_index_0, edge_index_1,
         W0, al0, ar0, W1, al1, ar1, sW1, sb1, sW2):
    W = jnp.stack([W0, W1])
    al = jnp.stack([al0, al1])
    ar = jnp.stack([ar0, ar1])
    eye = jnp.eye(H, dtype=jnp.float32)
    ALb = (al[:, :, :, None] * eye[None, :, None, :]).reshape(P, HF, H)
    ARb = (ar[:, :, :, None] * eye[None, :, None, :]).reshape(P, HF, H)
    pad = jnp.zeros((P, HF, 112), jnp.float32)
    ALR = jnp.concatenate([ALb, ARb, pad], axis=-1)

    h4, elr = _stage1(feat, W, ALR)

    src0 = edge_index_0[0]
    dst0 = edge_index_0[1]
    src1 = edge_index_1[0]
    dst1 = edge_index_1[1]

    ee = _stats(elr[0], elr[1], src0, dst0, src1, dst1)
    src2d = jnp.concatenate([src0, src1]).reshape(2 * E // ECH, ECH)
    dst2d = jnp.concatenate([dst0, dst1]).reshape(2 * E // ECH, ECH)
    ones = jnp.ones((N, 128), jnp.float32)
    h5 = jnp.concatenate([h4[0].reshape(4 * N, 128), ones,
                          h4[1].reshape(4 * N, 128), ones])
    raw = _agg(h5, ee.reshape(P * E, 16), src2d, dst2d)

    SEL = np.zeros((4, 128, 128), np.float32)
    for hp in range(4):
        SEL[hp, 16 * (2 * hp), 0:64] = 1.0
        SEL[hp, 16 * (2 * hp + 1), 64:128] = 1.0
    z, wsum = _c1(raw.reshape(P, 5, N, 128), sW1,
                  sb1.reshape(1, 128), sW2, jnp.asarray(SEL))
    return _c2(z, wsum)


def kernel(feat, edge_index_0, edge_index_1, edge_idx,
           W0, al0, ar0, W1, al1, ar1, sW1, sb1, sW2):
    del edge_idx
    return _run(feat, edge_index_0, edge_index_1,
                W0, al0, ar0, W1, al1, ar1, sW1, sb1, sW2)


# speedup vs baseline: 1.1580x; 1.1580x over previous
import functools

import jax
import jax.numpy as jnp
import numpy as np
from jax import lax
from jax.experimental import pallas as pl
from jax.experimental.pallas import tpu as pltpu
from jax.experimental.pallas import tpu_sc as plsc

N = 10000
E = 320000
D = 128
H = 8
F = 64
HF = 512
P = 2
NC, NS, L = 2, 16, 16
NW = NC * NS

BN = 1000
NB = N // BN

ECH = 128
KSUB = 10
BIG = ECH * KSUB
NBC = E // BIG
ER2 = E // ECH


def _proj_body(feat_ref, w_ref, alr_ref, h4_ref, elr_ref):
    f = feat_ref[...]
    w = w_ref[0]
    for hp in range(4):
        h4_ref[0, hp] = jnp.dot(f, w[:, hp * 128:(hp + 1) * 128],
                                preferred_element_type=jnp.float32)
    wlr = jnp.dot(w, alr_ref[0], preferred_element_type=jnp.float32)
    elr_ref[0] = jnp.dot(f, wlr, preferred_element_type=jnp.float32)


def _stage1(feat, W, ALR):
    return pl.pallas_call(
        _proj_body,
        grid=(P, NB),
        in_specs=[
            pl.BlockSpec((BN, D), lambda p, nb: (nb, 0)),
            pl.BlockSpec((1, D, HF), lambda p, nb: (p, 0, 0)),
            pl.BlockSpec((1, HF, 128), lambda p, nb: (p, 0, 0)),
        ],
        out_specs=[
            pl.BlockSpec((1, 4, BN, 128), lambda p, nb: (p, 0, nb, 0)),
            pl.BlockSpec((1, BN, 128), lambda p, nb: (p, nb, 0)),
        ],
        out_shape=[
            jax.ShapeDtypeStruct((P, 4, N, 128), jnp.float32),
            jax.ShapeDtypeStruct((P, N, 128), jnp.float32),
        ],
    )(feat, W, ALR)


_MESH = plsc.VectorSubcoreMesh(core_axis_name="c", subcore_axis_name="s",
                               num_cores=NC, num_subcores=NS)
_MESH1 = plsc.VectorSubcoreMesh(core_axis_name="c", subcore_axis_name="s",
                                num_cores=1, num_subcores=NS)


def _row_split(s):
    start = jnp.where(s < 2, s * 632, 1264 + (s - 2) * 624)
    return pl.multiple_of(start, 8)


def _load_idx(arr, base, buf, sem):
    cps = [pltpu.async_copy(arr.at[pl.ds(base + r * ECH, ECH)],
                            buf.at[r], sem) for r in range(KSUB)]
    for cp in cps:
        cp.wait()


def _stats_body(elr0_h, elr1_h, src0, dst0, src1, dst1, ee_hbm,
                srcv, dstv, gs, wl, wr, semg, semg2, semi):
    s = lax.axis_index("s")
    elrs = (elr0_h, elr1_h)
    srcs = (src0, src1)
    dsts = (dst0, dst1)

    nbc = jnp.where(s < (NBC % NS), NBC // NS + 1, NBC // NS)
    for p in range(P):

        def _chunk(k, _):
            bc = s + k * NS
            base = pl.multiple_of(bc * BIG, 8)
            _load_idx(srcs[p], base, srcv, semi)
            _load_idx(dsts[p], base, dstv, semi)

            for r in range(KSUB):
                ca = pltpu.async_copy(elrs[p].at[srcv.at[r]], wl, semg)
                cb = pltpu.async_copy(elrs[p].at[dstv.at[r]], wr, semg2)
                ca.wait()
                cb.wait()

                @plsc.parallel_loop(0, ECH, unroll=2)
                def _cmp(i):
                    x = wl[i, pl.ds(0, 16)] + wr[i, pl.ds(8, 16)]
                    gs[r * ECH + i, :] = jnp.exp(jnp.maximum(x, 0.2 * x))

            pltpu.sync_copy(gs, ee_hbm.at[p, pl.ds(base, BIG)])
            return 0
        lax.fori_loop(0, nbc, _chunk, 0)


def _stats(elr0, elr1, src0, dst0, src1, dst1):
    kk = pl.kernel(
        _stats_body,
        out_type=jax.ShapeDtypeStruct((P, E, 16), jnp.float32),
        mesh=_MESH1,
        scratch_types=[
            pltpu.VMEM((KSUB, ECH), jnp.int32),
            pltpu.VMEM((KSUB, ECH), jnp.int32),
            pltpu.VMEM((BIG, 16), jnp.float32),
            pltpu.VMEM((ECH, 128), jnp.float32),
            pltpu.VMEM((ECH, 128), jnp.float32),
            pltpu.SemaphoreType.DMA,
            pltpu.SemaphoreType.DMA,
            pltpu.SemaphoreType.DMA,
        ],
        compiler_params=pltpu.CompilerParams(use_tc_tiling_on_sc=False),
    )
    return kk(elr0, elr1, src0, dst0, src1, dst1)


def _agg_body(h5f, eef, src2d, dst2d, raw_hbm,
              srcv, dstv, eeb, hb, hb2, zb, acc,
              semg, semg2, sems, sems2, semi):
    s = lax.axis_index("s")

    tstart = pl.multiple_of(jnp.where(s == 0, 0, 320 + (s - 1) * 312), 8)

    def _z(i, _):
        for v in range(8):
            zb[i, pl.ds(v * L, L)] = jnp.zeros((L,), jnp.float32)
        return 0
    lax.fori_loop(0, 104, _z, 0)

    nbc = jnp.where(s < (NBC % NS), NBC // NS + 1, NBC // NS)

    for hp in range(5):
        col0 = 2 * hp

        def _round(p, _):
            hbase = (p * 5 + hp) * N
            for m in range(2):
                mbase = m * 5000

                for q in range(3):
                    off = pl.multiple_of(q * 104, 8)
                    pltpu.sync_copy(zb, acc.at[pl.ds(tstart + off, 104)])

                @pl.when(s == 0)
                def _():
                    pltpu.sync_copy(zb.at[pl.ds(0, 8)],
                                    acc.at[pl.ds(312, 8)])
                plsc.subcore_barrier()

                def _chunk(k, _):
                    bc = s + k * NS
                    erow = p * (E // ECH) + bc * KSUB
                    ebase = pl.multiple_of(p * E + bc * BIG, 8)
                    c1 = pltpu.async_copy(src2d.at[pl.ds(erow, KSUB)],
                                          srcv, semi)
                    c2 = pltpu.async_copy(dst2d.at[pl.ds(erow, KSUB)],
                                          dstv, semi)
                    c3 = pltpu.async_copy(eef.at[pl.ds(ebase, BIG)],
                                          eeb, semi)
                    c1.wait()
                    c2.wait()
                    c3.wait()
                    for r in range(KSUB):
                        for v in range(ECH // L):
                            sl = pl.ds(v * L, L)
                            srcv[r, sl] = srcv[r, sl] + hbase
                            t = dstv[r, sl] - mbase
                            ok = (t >= 0) & (t < 5000)
                            dstv[r, sl] = jnp.where(ok, t, 5000)

                    hbs = (hb, hb2)
                    sgs = (semg, semg2)
                    sss = (sems, sems2)
                    if hp < 4:
                        pltpu.async_copy(h5f.at[srcv.at[0]], hb, sgs[0])
                    for r in range(KSUB):
                        b = hbs[r % 2]
                        if hp < 4:
                            if r + 1 < KSUB:
                                ob = hbs[(r + 1) % 2]
                                if r >= 1:
                                    pltpu.make_async_copy(
                                        ob, acc.at[dstv.at[r - 1]],
                                        sss[(r + 1) % 2]).wait()
                                pltpu.async_copy(h5f.at[srcv.at[r + 1]],
                                                 ob, sgs[(r + 1) % 2])
                            pltpu.make_async_copy(h5f.at[srcv.at[r]],
                                                  b, sgs[r % 2]).wait()

                            @plsc.parallel_loop(0, ECH, unroll=2)
                            def _scale(i):
                                ev = eeb[r * ECH + i, :]
                                s0 = ev[col0]
                                s1 = ev[col0 + 1]
                                for v in range(4):
                                    b[i, pl.ds(v * L, L)] = (
                                        b[i, pl.ds(v * L, L)] * s0)
                                for v in range(4, 8):
                                    b[i, pl.ds(v * L, L)] = (
                                        b[i, pl.ds(v * L, L)] * s1)
                        else:
                            @plsc.parallel_loop(0, ECH, unroll=1)
                            def _scale(i):
                                ev = eeb[r * ECH + i, :]
                                for v in range(8):
                                    b[i, pl.ds(v * L, L)] = (
                                        jnp.ones((L,), jnp.float32) * ev[v])

                        cp = pltpu.async_copy(b, acc.at[dstv.at[r]],
                                              sss[r % 2], add=True)
                        if hp >= 4 or r >= KSUB - 2:
                            cp.wait()
                    return 0
                lax.fori_loop(0, nbc, _chunk, 0)

                plsc.subcore_barrier()
                rbase = pl.multiple_of(hbase + mbase + tstart, 8)
                for q in range(3):
                    off = pl.multiple_of(q * 104, 8)
                    pltpu.sync_copy(acc.at[pl.ds(tstart + off, 104)],
                                    raw_hbm.at[pl.ds(rbase + off, 104)])

                @pl.when(s == 0)
                def _():
                    pltpu.sync_copy(acc.at[pl.ds(312, 8)],
                                    raw_hbm.at[pl.ds(rbase + 312, 8)])
            return 0
        lax.fori_loop(0, P, _round, 0)


def _agg(h5f, eef, src2d, dst2d):
    kk = pl.kernel(
        _agg_body,
        out_type=jax.ShapeDtypeStruct((P * 5 * N, 128), jnp.float32),
        mesh=_MESH1,
        scratch_types=[
            pltpu.VMEM((KSUB, ECH), jnp.int32),
            pltpu.VMEM((KSUB, ECH), jnp.int32),
            pltpu.VMEM((BIG, 16), jnp.float32),
            pltpu.VMEM((ECH, 128), jnp.float32),
            pltpu.VMEM((ECH, 128), jnp.float32),
            pltpu.VMEM((104, 128), jnp.float32),
            pltpu.VMEM_SHARED((5008, 128), jnp.float32),
            pltpu.SemaphoreType.DMA,
            pltpu.SemaphoreType.DMA,
            pltpu.SemaphoreType.DMA,
            pltpu.SemaphoreType.DMA,
            pltpu.SemaphoreType.DMA,
        ],
        compiler_params=pltpu.CompilerParams(use_tc_tiling_on_sc=False),
    )
    return kk(h5f, eef, src2d, dst2d)


def _c1_body(raw_ref, sw1_ref, sb1_ref, sw2_ref, sel_ref,
             z_ref, wsum_ref):
    p = pl.program_id(0)
    nb = pl.program_id(1)
    d = raw_ref[0, 4]
    wacc = jnp.zeros((BN, 128), jnp.float32)
    for hp in range(4):
        den = jnp.dot(d, sel_ref[hp],
                      preferred_element_type=jnp.float32) + 1e-9
        zhp = raw_ref[0, hp] / den
        zhp = jnp.where(zhp > 0, zhp, jnp.exp(jnp.minimum(zhp, 0.0)) - 1.0)
        z_ref[0, :, hp * 128:(hp + 1) * 128] = zhp
        wacc = wacc + jnp.dot(zhp, sw1_ref[hp * 128:(hp + 1) * 128, :],
                              preferred_element_type=jnp.float32)
    w = jnp.dot(jnp.tanh(wacc + sb1_ref[...]), sw2_ref[...],
                preferred_element_type=jnp.float32)
    sc = jnp.sum(w)

    @pl.when(nb == 0)
    def _():
        wsum_ref[p, 0] = 0.0

    wsum_ref[p, 0] = wsum_ref[p, 0] + sc


def _c1(raw4, sW1, sb1r, sW2, SEL):
    return pl.pallas_call(
        _c1_body,
        grid=(P, NB),
        in_specs=[
            pl.BlockSpec((1, 5, BN, 128), lambda p, nb: (p, 0, nb, 0)),
            pl.BlockSpec((HF, 128), lambda p, nb: (0, 0)),
            pl.BlockSpec((1, 128), lambda p, nb: (0, 0)),
            pl.BlockSpec((128, 1), lambda p, nb: (0, 0)),
            pl.BlockSpec((4, 128, 128), lambda p, nb: (0, 0, 0)),
        ],
        out_specs=[
            pl.BlockSpec((1, BN, HF), lambda p, nb: (p, nb, 0)),
            pl.BlockSpec((P, 1), lambda p, nb: (0, 0),
                         memory_space=pltpu.SMEM),
        ],
        out_shape=[
            jax.ShapeDtypeStruct((P, N, HF), jnp.float32),
            jax.ShapeDtypeStruct((P, 1), jnp.float32),
        ],
    )(raw4, sW1, sb1r, sW2, SEL)


def _c2_body(z_ref, wsum_ref, out_ref):
    w0 = wsum_ref[0, 0] / N
    w1 = wsum_ref[1, 0] / N
    m = jnp.maximum(w0, w1)
    b0 = jnp.exp(w0 - m)
    b1 = jnp.exp(w1 - m)
    t = b0 + b1
    out_ref[...] = (b0 / t) * z_ref[0] + (b1 / t) * z_ref[1]


def _c2(z, wsum):
    return pl.pallas_call(
        _c2_body,
        grid=(NB,),
        in_specs=[
            pl.BlockSpec((P, BN, HF), lambda nb: (0, nb, 0)),
            pl.BlockSpec((P, 1), lambda nb: (0, 0), memory_space=pltpu.SMEM),
        ],
        out_specs=pl.BlockSpec((BN, HF), lambda nb: (nb, 0)),
        out_shape=jax.ShapeDtypeStruct((N, HF), jnp.float32),
    )(z, wsum)


@jax.jit
def _run(feat, edge_index_0, edge_index_1,
         W0, al0, ar0, W1, al1, ar1, sW1, sb1, sW2):
    W = jnp.stack([W0, W1])
    al = jnp.stack([al0, al1])
    ar = jnp.stack([ar0, ar1])
    eye = jnp.eye(H, dtype=jnp.float32)
    ALb = (al[:, :, :, None] * eye[None, :, None, :]).reshape(P, HF, H)
    ARb = (ar[:, :, :, None] * eye[None, :, None, :]).reshape(P, HF, H)
    pad = jnp.zeros((P, HF, 112), jnp.float32)
    ALR = jnp.concatenate([ALb, ARb, pad], axis=-1)

    h4, elr = _stage1(feat, W, ALR)

    src0 = edge_index_0[0]
    dst0 = edge_index_0[1]
    src1 = edge_index_1[0]
    dst1 = edge_index_1[1]

    ee = _stats(elr[0], elr[1], src0, dst0, src1, dst1)
    src2d = jnp.concatenate([src0, src1]).reshape(2 * E // ECH, ECH)
    dst2d = jnp.concatenate([dst0, dst1]).reshape(2 * E // ECH, ECH)
    ones = jnp.ones((N, 128), jnp.float32)
    h5 = jnp.concatenate([h4[0].reshape(4 * N, 128), ones,
                          h4[1].reshape(4 * N, 128), ones])
    raw = _agg(h5, ee.reshape(P * E, 16), src2d, dst2d)

    SEL = np.zeros((4, 128, 128), np.float32)
    for hp in range(4):
        SEL[hp, 16 * (2 * hp), 0:64] = 1.0
        SEL[hp, 16 * (2 * hp + 1), 64:128] = 1.0
    z, wsum = _c1(raw.reshape(P, 5, N, 128), sW1,
                  sb1.reshape(1, 128), sW2, jnp.asarray(SEL))
    return _c2(z, wsum)


def kernel(feat, edge_index_0, edge_index_1, edge_idx,
           W0, al0, ar0, W1, al1, ar1, sW1, sb1, sW2):
    del edge_idx
    return _run(feat, edge_index_0, edge_index_1,
                W0, al0, ar0, W1, al1, ar1, sW1, sb1, sW2)


# speedup vs baseline: 1.3274x; 1.1464x over previous
import functools

import jax
import jax.numpy as jnp
import numpy as np
from jax import lax
from jax.experimental import pallas as pl
from jax.experimental.pallas import tpu as pltpu
from jax.experimental.pallas import tpu_sc as plsc

N = 10000
E = 320000
D = 128
H = 8
F = 64
HF = 512
P = 2
NC, NS, L = 2, 16, 16
NW = NC * NS

BN = 1000
NB = N // BN

ECH = 128
KSUB = 10
BIG = ECH * KSUB
NBC = E // BIG
ER2 = E // ECH


def _proj_body(feat_ref, w_ref, alr_ref, h4_ref, elr_ref):
    f = feat_ref[...]
    w = w_ref[0]
    for hp in range(4):
        h4_ref[0, hp] = jnp.dot(f, w[:, hp * 128:(hp + 1) * 128],
                                preferred_element_type=jnp.float32)
    wlr = jnp.dot(w, alr_ref[0], preferred_element_type=jnp.float32)
    elr_ref[0] = jnp.dot(f, wlr, preferred_element_type=jnp.float32)


def _stage1(feat, W, ALR):
    return pl.pallas_call(
        _proj_body,
        grid=(P, NB),
        in_specs=[
            pl.BlockSpec((BN, D), lambda p, nb: (nb, 0)),
            pl.BlockSpec((1, D, HF), lambda p, nb: (p, 0, 0)),
            pl.BlockSpec((1, HF, 128), lambda p, nb: (p, 0, 0)),
        ],
        out_specs=[
            pl.BlockSpec((1, 4, BN, 128), lambda p, nb: (p, 0, nb, 0)),
            pl.BlockSpec((1, BN, 128), lambda p, nb: (p, nb, 0)),
        ],
        out_shape=[
            jax.ShapeDtypeStruct((P, 4, N, 128), jnp.float32),
            jax.ShapeDtypeStruct((P, N, 128), jnp.float32),
        ],
    )(feat, W, ALR)


_MESH = plsc.VectorSubcoreMesh(core_axis_name="c", subcore_axis_name="s",
                               num_cores=NC, num_subcores=NS)
_MESH1 = plsc.VectorSubcoreMesh(core_axis_name="c", subcore_axis_name="s",
                                num_cores=1, num_subcores=NS)


def _row_split(s):
    start = jnp.where(s < 2, s * 632, 1264 + (s - 2) * 624)
    return pl.multiple_of(start, 8)


def _load_idx(arr, base, buf, sem):
    cps = [pltpu.async_copy(arr.at[pl.ds(base + r * ECH, ECH)],
                            buf.at[r], sem) for r in range(KSUB)]
    for cp in cps:
        cp.wait()


def _stats_body(elr0_h, elr1_h, src0, dst0, src1, dst1, ee_hbm, dpart_hbm,
                srcv, dstv, gs, zb, wl, wr, dflat, semg, semg2, sems, semi):
    s = lax.axis_index("s")
    elrs = (elr0_h, elr1_h)
    srcs = (src0, src1)
    dsts = (dst0, dst1)
    start = _row_split(s)

    def _z(i, _):
        zb[i, :] = jnp.zeros((16,), jnp.float32)
        return 0
    lax.fori_loop(0, 632, _z, 0)

    nbc = jnp.where(s < (NBC % NS), NBC // NS + 1, NBC // NS)
    for p in range(P):

        @pl.when(s < 2)
        def _():
            pltpu.sync_copy(zb, dflat.at[pl.ds(start, 632)])

        @pl.when(s >= 2)
        def _():
            pltpu.sync_copy(zb.at[pl.ds(0, 624)], dflat.at[pl.ds(start, 624)])
        plsc.subcore_barrier()

        def _chunk(k, _):
            bc = s + k * NS
            base = pl.multiple_of(bc * BIG, 8)
            _load_idx(srcs[p], base, srcv, semi)
            _load_idx(dsts[p], base, dstv, semi)

            for r in range(KSUB):
                ca = pltpu.async_copy(elrs[p].at[srcv.at[r]], wl, semg)
                cb = pltpu.async_copy(elrs[p].at[dstv.at[r]], wr, semg2)
                ca.wait()
                cb.wait()

                @plsc.parallel_loop(0, ECH, unroll=2)
                def _cmp(i):
                    x = wl[i, pl.ds(0, 16)] + wr[i, pl.ds(8, 16)]
                    gs[r * ECH + i, :] = jnp.exp(jnp.maximum(x, 0.2 * x))

            pltpu.sync_copy(gs, ee_hbm.at[p, pl.ds(base, BIG)])
            for r in range(KSUB):
                pltpu.async_copy(gs.at[pl.ds(r * ECH, ECH)],
                                 dflat.at[dstv.at[r]], sems, add=True).wait()
            return 0
        lax.fori_loop(0, nbc, _chunk, 0)
        plsc.subcore_barrier()

        def _emit(nq, tail):
            for q in range(nq):
                off = pl.multiple_of(q * 104, 8)
                pltpu.sync_copy(dflat.at[pl.ds(start + off, 104)],
                                gs.at[pl.ds(0, 104)])

                def _exp(i, _):
                    wl[i, pl.ds(0, 16)] = gs[i, :]
                    return 0
                lax.fori_loop(0, 104, _exp, 0)
                db = pl.multiple_of(p * N + start + off, 8)
                pltpu.sync_copy(wl.at[pl.ds(0, 104)],
                                dpart_hbm.at[pl.ds(db, 104)])
            if tail:
                off = pl.multiple_of(624, 8)
                pltpu.sync_copy(dflat.at[pl.ds(start + off, 8)],
                                gs.at[pl.ds(0, 8)])

                def _exp2(i, _):
                    wl[i, pl.ds(0, 16)] = gs[i, :]
                    return 0
                lax.fori_loop(0, 8, _exp2, 0)
                db = pl.multiple_of(p * N + start + off, 8)
                pltpu.sync_copy(wl.at[pl.ds(0, 8)],
                                dpart_hbm.at[pl.ds(db, 8)])

        @pl.when(s < 2)
        def _():
            _emit(6, True)

        @pl.when(s >= 2)
        def _():
            _emit(6, False)


def _stats(elr0, elr1, src0, dst0, src1, dst1):
    kk = pl.kernel(
        _stats_body,
        out_type=(
            jax.ShapeDtypeStruct((P, E, 16), jnp.float32),
            jax.ShapeDtypeStruct((P * N, 128), jnp.float32),
        ),
        mesh=_MESH1,
        scratch_types=[
            pltpu.VMEM((KSUB, ECH), jnp.int32),
            pltpu.VMEM((KSUB, ECH), jnp.int32),
            pltpu.VMEM((BIG, 16), jnp.float32),
            pltpu.VMEM((632, 16), jnp.float32),
            pltpu.VMEM((ECH, 128), jnp.float32),
            pltpu.VMEM((ECH, 128), jnp.float32),
            pltpu.VMEM_SHARED((N, 16), jnp.float32),
            pltpu.SemaphoreType.DMA,
            pltpu.SemaphoreType.DMA,
            pltpu.SemaphoreType.DMA,
            pltpu.SemaphoreType.DMA,
        ],
        compiler_params=pltpu.CompilerParams(use_tc_tiling_on_sc=False),
    )
    return kk(elr0, elr1, src0, dst0, src1, dst1)


def _agg_body(h5f, eef, src2d, dst2d, raw_hbm,
              srcv, dstv, eeb, hb, hb2, zb, acc,
              semg, semg2, sems, sems2, semi):
    s = lax.axis_index("s")

    tstart = pl.multiple_of(jnp.where(s == 0, 0, 320 + (s - 1) * 312), 8)

    def _z(i, _):
        for v in range(8):
            zb[i, pl.ds(v * L, L)] = jnp.zeros((L,), jnp.float32)
        return 0
    lax.fori_loop(0, 104, _z, 0)

    nbc = jnp.where(s < (NBC % NS), NBC // NS + 1, NBC // NS)

    for hp in range(4):
        col0 = 2 * hp

        def _round(p, _):
            hbase = (p * 4 + hp) * N
            for m in range(2):
                mbase = m * 5000

                for q in range(3):
                    off = pl.multiple_of(q * 104, 8)
                    pltpu.sync_copy(zb, acc.at[pl.ds(tstart + off, 104)])

                @pl.when(s == 0)
                def _():
                    pltpu.sync_copy(zb.at[pl.ds(0, 8)],
                                    acc.at[pl.ds(312, 8)])
                plsc.subcore_barrier()

                def _chunk(k, _):
                    bc = s + k * NS
                    erow = p * (E // ECH) + bc * KSUB
                    ebase = pl.multiple_of(p * E + bc * BIG, 8)
                    c1 = pltpu.async_copy(src2d.at[pl.ds(erow, KSUB)],
                                          srcv, semi)
                    c2 = pltpu.async_copy(dst2d.at[pl.ds(erow, KSUB)],
                                          dstv, semi)
                    c3 = pltpu.async_copy(eef.at[pl.ds(ebase, BIG)],
                                          eeb, semi)
                    c1.wait()
                    c2.wait()
                    c3.wait()
                    for r in range(KSUB):
                        for v in range(ECH // L):
                            sl = pl.ds(v * L, L)
                            srcv[r, sl] = srcv[r, sl] + hbase
                            t = dstv[r, sl] - mbase
                            ok = (t >= 0) & (t < 5000)
                            dstv[r, sl] = jnp.where(ok, t, 5000)

                    hbs = (hb, hb2)
                    sgs = (semg, semg2)
                    sss = (sems, sems2)
                    pltpu.async_copy(h5f.at[srcv.at[0]], hb, sgs[0])
                    for r in range(KSUB):
                        b = hbs[r % 2]
                        if True:
                            if r + 1 < KSUB:
                                ob = hbs[(r + 1) % 2]
                                if r >= 1:
                                    pltpu.make_async_copy(
                                        ob, acc.at[dstv.at[r - 1]],
                                        sss[(r + 1) % 2]).wait()
                                pltpu.async_copy(h5f.at[srcv.at[r + 1]],
                                                 ob, sgs[(r + 1) % 2])
                            pltpu.make_async_copy(h5f.at[srcv.at[r]],
                                                  b, sgs[r % 2]).wait()

                            @plsc.parallel_loop(0, ECH, unroll=2)
                            def _scale(i):
                                ev = eeb[r * ECH + i, :]
                                s0 = ev[col0]
                                s1 = ev[col0 + 1]
                                for v in range(4):
                                    b[i, pl.ds(v * L, L)] = (
                                        b[i, pl.ds(v * L, L)] * s0)
                                for v in range(4, 8):
                                    b[i, pl.ds(v * L, L)] = (
                                        b[i, pl.ds(v * L, L)] * s1)
                        cp = pltpu.async_copy(b, acc.at[dstv.at[r]],
                                              sss[r % 2], add=True)
                        if r >= KSUB - 2:
                            cp.wait()
                    return 0
                lax.fori_loop(0, nbc, _chunk, 0)

                plsc.subcore_barrier()
                rbase = pl.multiple_of(hbase + mbase + tstart, 8)
                for q in range(3):
                    off = pl.multiple_of(q * 104, 8)
                    pltpu.sync_copy(acc.at[pl.ds(tstart + off, 104)],
                                    raw_hbm.at[pl.ds(rbase + off, 104)])

                @pl.when(s == 0)
                def _():
                    pltpu.sync_copy(acc.at[pl.ds(312, 8)],
                                    raw_hbm.at[pl.ds(rbase + 312, 8)])
            return 0
        lax.fori_loop(0, P, _round, 0)


def _agg(h5f, eef, src2d, dst2d):
    kk = pl.kernel(
        _agg_body,
        out_type=jax.ShapeDtypeStruct((P * 4 * N, 128), jnp.float32),
        mesh=_MESH1,
        scratch_types=[
            pltpu.VMEM((KSUB, ECH), jnp.int32),
            pltpu.VMEM((KSUB, ECH), jnp.int32),
            pltpu.VMEM((BIG, 16), jnp.float32),
            pltpu.VMEM((ECH, 128), jnp.float32),
            pltpu.VMEM((ECH, 128), jnp.float32),
            pltpu.VMEM((104, 128), jnp.float32),
            pltpu.VMEM_SHARED((5008, 128), jnp.float32),
            pltpu.SemaphoreType.DMA,
            pltpu.SemaphoreType.DMA,
            pltpu.SemaphoreType.DMA,
            pltpu.SemaphoreType.DMA,
            pltpu.SemaphoreType.DMA,
        ],
        compiler_params=pltpu.CompilerParams(use_tc_tiling_on_sc=False),
    )
    return kk(h5f, eef, src2d, dst2d)


def _c1_body(raw_ref, dpart_ref, sw1_ref, sb1_ref, sw2_ref, sel_ref,
             z_ref, wsum_ref):
    p = pl.program_id(0)
    nb = pl.program_id(1)
    d = dpart_ref[0]
    wacc = jnp.zeros((BN, 128), jnp.float32)
    for hp in range(4):
        den = jnp.dot(d, sel_ref[hp],
                      preferred_element_type=jnp.float32) + 1e-9
        zhp = raw_ref[0, hp] / den
        zhp = jnp.where(zhp > 0, zhp, jnp.exp(jnp.minimum(zhp, 0.0)) - 1.0)
        z_ref[0, :, hp * 128:(hp + 1) * 128] = zhp
        wacc = wacc + jnp.dot(zhp, sw1_ref[hp * 128:(hp + 1) * 128, :],
                              preferred_element_type=jnp.float32)
    w = jnp.dot(jnp.tanh(wacc + sb1_ref[...]), sw2_ref[...],
                preferred_element_type=jnp.float32)
    sc = jnp.sum(w)

    @pl.when(nb == 0)
    def _():
        wsum_ref[p, 0] = 0.0

    wsum_ref[p, 0] = wsum_ref[p, 0] + sc


def _c1(raw4, dpart4, sW1, sb1r, sW2, SEL):
    return pl.pallas_call(
        _c1_body,
        grid=(P, NB),
        in_specs=[
            pl.BlockSpec((1, 4, BN, 128), lambda p, nb: (p, 0, nb, 0)),
            pl.BlockSpec((1, BN, 128), lambda p, nb: (p, nb, 0)),
            pl.BlockSpec((HF, 128), lambda p, nb: (0, 0)),
            pl.BlockSpec((1, 128), lambda p, nb: (0, 0)),
            pl.BlockSpec((128, 1), lambda p, nb: (0, 0)),
            pl.BlockSpec((4, 128, 128), lambda p, nb: (0, 0, 0)),
        ],
        out_specs=[
            pl.BlockSpec((1, BN, HF), lambda p, nb: (p, nb, 0)),
            pl.BlockSpec((P, 1), lambda p, nb: (0, 0),
                         memory_space=pltpu.SMEM),
        ],
        out_shape=[
            jax.ShapeDtypeStruct((P, N, HF), jnp.float32),
            jax.ShapeDtypeStruct((P, 1), jnp.float32),
        ],
    )(raw4, dpart4, sW1, sb1r, sW2, SEL)


def _c2_body(z_ref, wsum_ref, out_ref):
    w0 = wsum_ref[0, 0] / N
    w1 = wsum_ref[1, 0] / N
    m = jnp.maximum(w0, w1)
    b0 = jnp.exp(w0 - m)
    b1 = jnp.exp(w1 - m)
    t = b0 + b1
    out_ref[...] = (b0 / t) * z_ref[0] + (b1 / t) * z_ref[1]


def _c2(z, wsum):
    return pl.pallas_call(
        _c2_body,
        grid=(NB,),
        in_specs=[
            pl.BlockSpec((P, BN, HF), lambda nb: (0, nb, 0)),
            pl.BlockSpec((P, 1), lambda nb: (0, 0), memory_space=pltpu.SMEM),
        ],
        out_specs=pl.BlockSpec((BN, HF), lambda nb: (nb, 0)),
        out_shape=jax.ShapeDtypeStruct((N, HF), jnp.float32),
    )(z, wsum)


@jax.jit
def _run(feat, edge_index_0, edge_index_1,
         W0, al0, ar0, W1, al1, ar1, sW1, sb1, sW2):
    W = jnp.stack([W0, W1])
    al = jnp.stack([al0, al1])
    ar = jnp.stack([ar0, ar1])
    eye = jnp.eye(H, dtype=jnp.float32)
    ALb = (al[:, :, :, None] * eye[None, :, None, :]).reshape(P, HF, H)
    ARb = (ar[:, :, :, None] * eye[None, :, None, :]).reshape(P, HF, H)
    pad = jnp.zeros((P, HF, 112), jnp.float32)
    ALR = jnp.concatenate([ALb, ARb, pad], axis=-1)

    h4, elr = _stage1(feat, W, ALR)

    src0 = edge_index_0[0]
    dst0 = edge_index_0[1]
    src1 = edge_index_1[0]
    dst1 = edge_index_1[1]

    ee, dpart = _stats(elr[0], elr[1], src0, dst0, src1, dst1)
    src2d = jnp.concatenate([src0, src1]).reshape(2 * E // ECH, ECH)
    dst2d = jnp.concatenate([dst0, dst1]).reshape(2 * E // ECH, ECH)
    raw = _agg(h4.reshape(P * 4 * N, 128), ee.reshape(P * E, 16),
               src2d, dst2d)

    SEL = np.zeros((4, 128, 128), np.float32)
    for hp in range(4):
        SEL[hp, 2 * hp, 0:64] = 1.0
        SEL[hp, 2 * hp + 1, 64:128] = 1.0
    z, wsum = _c1(raw.reshape(P, 4, N, 128), dpart.reshape(P, N, 128),
                  sW1, sb1.reshape(1, 128), sW2, jnp.asarray(SEL))
    return _c2(z, wsum)


def kernel(feat, edge_index_0, edge_index_1, edge_idx,
           W0, al0, ar0, W1, al1, ar1, sW1, sb1, sW2):
    del edge_idx
    return _run(feat, edge_index_0, edge_index_1,
                W0, al0, ar0, W1, al1, ar1, sW1, sb1, sW2)


# speedup vs baseline: 1.3317x; 1.0032x over previous
import functools

import jax
import jax.numpy as jnp
import numpy as np
from jax import lax
from jax.experimental import pallas as pl
from jax.experimental.pallas import tpu as pltpu
from jax.experimental.pallas import tpu_sc as plsc

N = 10000
E = 320000
D = 128
H = 8
F = 64
HF = 512
P = 2
NC, NS, L = 2, 16, 16
NW = NC * NS

BN = 1000
NB = N // BN

ECH = 128
KSUB = 10
BIG = ECH * KSUB
NBC = E // BIG
ER2 = E // ECH


def _proj_body(feat_ref, w_ref, alr_ref, h4_ref, elr_ref):
    f = feat_ref[...]
    w = w_ref[0]
    for hp in range(4):
        h4_ref[0, hp] = jnp.dot(f, w[:, hp * 128:(hp + 1) * 128],
                                preferred_element_type=jnp.float32)
    wlr = jnp.dot(w, alr_ref[0], preferred_element_type=jnp.float32)
    elr_ref[0] = jnp.dot(f, wlr, preferred_element_type=jnp.float32)


def _stage1(feat, W, ALR):
    return pl.pallas_call(
        _proj_body,
        grid=(P, NB),
        in_specs=[
            pl.BlockSpec((BN, D), lambda p, nb: (nb, 0)),
            pl.BlockSpec((1, D, HF), lambda p, nb: (p, 0, 0)),
            pl.BlockSpec((1, HF, 128), lambda p, nb: (p, 0, 0)),
        ],
        out_specs=[
            pl.BlockSpec((1, 4, BN, 128), lambda p, nb: (p, 0, nb, 0)),
            pl.BlockSpec((1, BN, 128), lambda p, nb: (p, nb, 0)),
        ],
        out_shape=[
            jax.ShapeDtypeStruct((P, 4, N, 128), jnp.float32),
            jax.ShapeDtypeStruct((P, N, 128), jnp.float32),
        ],
    )(feat, W, ALR)


_MESH = plsc.VectorSubcoreMesh(core_axis_name="c", subcore_axis_name="s",
                               num_cores=NC, num_subcores=NS)
_MESH1 = plsc.VectorSubcoreMesh(core_axis_name="c", subcore_axis_name="s",
                                num_cores=1, num_subcores=NS)


def _row_split(s):
    start = jnp.where(s < 2, s * 632, 1264 + (s - 2) * 624)
    return pl.multiple_of(start, 8)


def _load_idx(arr, base, buf, sem):
    cps = [pltpu.async_copy(arr.at[pl.ds(base + r * ECH, ECH)],
                            buf.at[r], sem) for r in range(KSUB)]
    for cp in cps:
        cp.wait()


def _stats_body(elr0_h, elr1_h, src0, dst0, src1, dst1, ee_hbm, dpart_hbm,
                srcv, dstv, gs, zb, wl, wr, dflat, semg, semg2, sems, semi):
    s = lax.axis_index("s")
    elrs = (elr0_h, elr1_h)
    srcs = (src0, src1)
    dsts = (dst0, dst1)
    start = _row_split(s)

    def _z(i, _):
        zb[i, :] = jnp.zeros((16,), jnp.float32)
        return 0
    lax.fori_loop(0, 632, _z, 0)

    nbc = jnp.where(s < (NBC % NS), NBC // NS + 1, NBC // NS)
    for p in range(P):

        @pl.when(s < 2)
        def _():
            pltpu.sync_copy(zb, dflat.at[pl.ds(start, 632)])

        @pl.when(s >= 2)
        def _():
            pltpu.sync_copy(zb.at[pl.ds(0, 624)], dflat.at[pl.ds(start, 624)])
        plsc.subcore_barrier()

        def _chunk(k, _):
            bc = s + k * NS
            base = pl.multiple_of(bc * BIG, 8)
            _load_idx(srcs[p], base, srcv, semi)
            _load_idx(dsts[p], base, dstv, semi)

            for r in range(KSUB):
                ca = pltpu.async_copy(elrs[p].at[srcv.at[r]], wl, semg)
                cb = pltpu.async_copy(elrs[p].at[dstv.at[r]], wr, semg2)
                ca.wait()
                cb.wait()

                @plsc.parallel_loop(0, ECH, unroll=2)
                def _cmp(i):
                    x = wl[i, pl.ds(0, 16)] + wr[i, pl.ds(8, 16)]
                    gs[r * ECH + i, :] = jnp.exp(jnp.maximum(x, 0.2 * x))

            pltpu.sync_copy(gs, ee_hbm.at[p, pl.ds(base, BIG)])
            cps = [pltpu.async_copy(gs.at[pl.ds(r * ECH, ECH)],
                                    dflat.at[dstv.at[r]], sems, add=True)
                   for r in range(KSUB)]
            for cp in cps:
                cp.wait()
            return 0
        lax.fori_loop(0, nbc, _chunk, 0)
        plsc.subcore_barrier()

        def _emit(nq, tail):
            for q in range(nq):
                off = pl.multiple_of(q * 104, 8)
                pltpu.sync_copy(dflat.at[pl.ds(start + off, 104)],
                                gs.at[pl.ds(0, 104)])

                def _exp(i, _):
                    wl[i, pl.ds(0, 16)] = gs[i, :]
                    return 0
                lax.fori_loop(0, 104, _exp, 0)
                db = pl.multiple_of(p * N + start + off, 8)
                pltpu.sync_copy(wl.at[pl.ds(0, 104)],
                                dpart_hbm.at[pl.ds(db, 104)])
            if tail:
                off = pl.multiple_of(624, 8)
                pltpu.sync_copy(dflat.at[pl.ds(start + off, 8)],
                                gs.at[pl.ds(0, 8)])

                def _exp2(i, _):
                    wl[i, pl.ds(0, 16)] = gs[i, :]
                    return 0
                lax.fori_loop(0, 8, _exp2, 0)
                db = pl.multiple_of(p * N + start + off, 8)
                pltpu.sync_copy(wl.at[pl.ds(0, 8)],
                                dpart_hbm.at[pl.ds(db, 8)])

        @pl.when(s < 2)
        def _():
            _emit(6, True)

        @pl.when(s >= 2)
        def _():
            _emit(6, False)


def _stats(elr0, elr1, src0, dst0, src1, dst1):
    kk = pl.kernel(
        _stats_body,
        out_type=(
            jax.ShapeDtypeStruct((P, E, 16), jnp.float32),
            jax.ShapeDtypeStruct((P * N, 128), jnp.float32),
        ),
        mesh=_MESH1,
        scratch_types=[
            pltpu.VMEM((KSUB, ECH), jnp.int32),
            pltpu.VMEM((KSUB, ECH), jnp.int32),
            pltpu.VMEM((BIG, 16), jnp.float32),
            pltpu.VMEM((632, 16), jnp.float32),
            pltpu.VMEM((ECH, 128), jnp.float32),
            pltpu.VMEM((ECH, 128), jnp.float32),
            pltpu.VMEM_SHARED((N, 16), jnp.float32),
            pltpu.SemaphoreType.DMA,
            pltpu.SemaphoreType.DMA,
            pltpu.SemaphoreType.DMA,
            pltpu.SemaphoreType.DMA,
        ],
        compiler_params=pltpu.CompilerParams(use_tc_tiling_on_sc=False),
    )
    return kk(elr0, elr1, src0, dst0, src1, dst1)


def _agg_body(h5f, eef, src2d, dst2d, raw_hbm,
              srcv, dstv, eeb, hb, hb2, zb, acc,
              semg, semg2, sems, sems2, semi):
    s = lax.axis_index("s")

    tstart = pl.multiple_of(jnp.where(s == 0, 0, 320 + (s - 1) * 312), 8)

    def _z(i, _):
        for v in range(8):
            zb[i, pl.ds(v * L, L)] = jnp.zeros((L,), jnp.float32)
        return 0
    lax.fori_loop(0, 104, _z, 0)

    nbc = jnp.where(s < (NBC % NS), NBC // NS + 1, NBC // NS)

    for hp in range(4):
        col0 = 2 * hp

        def _round(p, _):
            hbase = (p * 4 + hp) * N
            for m in range(2):
                mbase = m * 5000

                for q in range(3):
                    off = pl.multiple_of(q * 104, 8)
                    pltpu.sync_copy(zb, acc.at[pl.ds(tstart + off, 104)])

                @pl.when(s == 0)
                def _():
                    pltpu.sync_copy(zb.at[pl.ds(0, 8)],
                                    acc.at[pl.ds(312, 8)])
                plsc.subcore_barrier()

                def _chunk(k, _):
                    bc = s + k * NS
                    erow = p * (E // ECH) + bc * KSUB
                    ebase = pl.multiple_of(p * E + bc * BIG, 8)
                    c1 = pltpu.async_copy(src2d.at[pl.ds(erow, KSUB)],
                                          srcv, semi)
                    c2 = pltpu.async_copy(dst2d.at[pl.ds(erow, KSUB)],
                                          dstv, semi)
                    c3 = pltpu.async_copy(eef.at[pl.ds(ebase, BIG)],
                                          eeb, semi)
                    c1.wait()
                    c2.wait()
                    c3.wait()
                    for r in range(KSUB):
                        for v in range(ECH // L):
                            sl = pl.ds(v * L, L)
                            srcv[r, sl] = srcv[r, sl] + hbase
                            t = dstv[r, sl] - mbase
                            ok = (t >= 0) & (t < 5000)
                            dstv[r, sl] = jnp.where(ok, t, 5000)

                    hbs = (hb, hb2)
                    sgs = (semg, semg2)
                    sss = (sems, sems2)
                    pltpu.async_copy(h5f.at[srcv.at[0]], hb, sgs[0])
                    for r in range(KSUB):
                        b = hbs[r % 2]
                        if True:
                            if r + 1 < KSUB:
                                ob = hbs[(r + 1) % 2]
                                if r >= 1:
                                    pltpu.make_async_copy(
                                        ob, acc.at[dstv.at[r - 1]],
                                        sss[(r + 1) % 2]).wait()
                                pltpu.async_copy(h5f.at[srcv.at[r + 1]],
                                                 ob, sgs[(r + 1) % 2])
                            pltpu.make_async_copy(h5f.at[srcv.at[r]],
                                                  b, sgs[r % 2]).wait()

                            @plsc.parallel_loop(0, ECH, unroll=2)
                            def _scale(i):
                                ev = eeb[r * ECH + i, :]
                                s0 = ev[col0]
                                s1 = ev[col0 + 1]
                                for v in range(4):
                                    b[i, pl.ds(v * L, L)] = (
                                        b[i, pl.ds(v * L, L)] * s0)
                                for v in range(4, 8):
                                    b[i, pl.ds(v * L, L)] = (
                                        b[i, pl.ds(v * L, L)] * s1)
                        cp = pltpu.async_copy(b, acc.at[dstv.at[r]],
                                              sss[r % 2], add=True)
                        if r >= KSUB - 2:
                            cp.wait()
                    return 0
                lax.fori_loop(0, nbc, _chunk, 0)

                plsc.subcore_barrier()
                rbase = pl.multiple_of(hbase + mbase + tstart, 8)
                for q in range(3):
                    off = pl.multiple_of(q * 104, 8)
                    pltpu.sync_copy(acc.at[pl.ds(tstart + off, 104)],
                                    raw_hbm.at[pl.ds(rbase + off, 104)])

                @pl.when(s == 0)
                def _():
                    pltpu.sync_copy(acc.at[pl.ds(312, 8)],
                                    raw_hbm.at[pl.ds(rbase + 312, 8)])
            return 0
        lax.fori_loop(0, P, _round, 0)


def _agg(h5f, eef, src2d, dst2d):
    kk = pl.kernel(
        _agg_body,
        out_type=jax.ShapeDtypeStruct((P * 4 * N, 128), jnp.float32),
        mesh=_MESH1,
        scratch_types=[
            pltpu.VMEM((KSUB, ECH), jnp.int32),
            pltpu.VMEM((KSUB, ECH), jnp.int32),
            pltpu.VMEM((BIG, 16), jnp.float32),
            pltpu.VMEM((ECH, 128), jnp.float32),
            pltpu.VMEM((ECH, 128), jnp.float32),
            pltpu.VMEM((104, 128), jnp.float32),
            pltpu.VMEM_SHARED((5008, 128), jnp.float32),
            pltpu.SemaphoreType.DMA,
            pltpu.SemaphoreType.DMA,
            pltpu.SemaphoreType.DMA,
            pltpu.SemaphoreType.DMA,
            pltpu.SemaphoreType.DMA,
        ],
        compiler_params=pltpu.CompilerParams(use_tc_tiling_on_sc=False),
    )
    return kk(h5f, eef, src2d, dst2d)


def _c1_body(raw_ref, dpart_ref, sw1_ref, sb1_ref, sw2_ref, sel_ref,
             z_ref, wsum_ref):
    p = pl.program_id(0)
    nb = pl.program_id(1)
    d = dpart_ref[0]
    wacc = jnp.zeros((BN, 128), jnp.float32)
    for hp in range(4):
        den = jnp.dot(d, sel_ref[hp],
                      preferred_element_type=jnp.float32) + 1e-9
        zhp = raw_ref[0, hp] / den
        zhp = jnp.where(zhp > 0, zhp, jnp.exp(jnp.minimum(zhp, 0.0)) - 1.0)
        z_ref[0, :, hp * 128:(hp + 1) * 128] = zhp
        wacc = wacc + jnp.dot(zhp, sw1_ref[hp * 128:(hp + 1) * 128, :],
                              preferred_element_type=jnp.float32)
    w = jnp.dot(jnp.tanh(wacc + sb1_ref[...]), sw2_ref[...],
                preferred_element_type=jnp.float32)
    sc = jnp.sum(w)

    @pl.when(nb == 0)
    def _():
        wsum_ref[p, 0] = 0.0

    wsum_ref[p, 0] = wsum_ref[p, 0] + sc


def _c1(raw4, dpart4, sW1, sb1r, sW2, SEL):
    return pl.pallas_call(
        _c1_body,
        grid=(P, NB),
        in_specs=[
            pl.BlockSpec((1, 4, BN, 128), lambda p, nb: (p, 0, nb, 0)),
            pl.BlockSpec((1, BN, 128), lambda p, nb: (p, nb, 0)),
            pl.BlockSpec((HF, 128), lambda p, nb: (0, 0)),
            pl.BlockSpec((1, 128), lambda p, nb: (0, 0)),
            pl.BlockSpec((128, 1), lambda p, nb: (0, 0)),
            pl.BlockSpec((4, 128, 128), lambda p, nb: (0, 0, 0)),
        ],
        out_specs=[
            pl.BlockSpec((1, BN, HF), lambda p, nb: (p, nb, 0)),
            pl.BlockSpec((P, 1), lambda p, nb: (0, 0),
                         memory_space=pltpu.SMEM),
        ],
        out_shape=[
            jax.ShapeDtypeStruct((P, N, HF), jnp.float32),
            jax.ShapeDtypeStruct((P, 1), jnp.float32),
        ],
    )(raw4, dpart4, sW1, sb1r, sW2, SEL)


def _c2_body(z_ref, wsum_ref, out_ref):
    w0 = wsum_ref[0, 0] / N
    w1 = wsum_ref[1, 0] / N
    m = jnp.maximum(w0, w1)
    b0 = jnp.exp(w0 - m)
    b1 = jnp.exp(w1 - m)
    t = b0 + b1
    out_ref[...] = (b0 / t) * z_ref[0] + (b1 / t) * z_ref[1]


def _c2(z, wsum):
    return pl.pallas_call(
        _c2_body,
        grid=(NB,),
        in_specs=[
            pl.BlockSpec((P, BN, HF), lambda nb: (0, nb, 0)),
            pl.BlockSpec((P, 1), lambda nb: (0, 0), memory_space=pltpu.SMEM),
        ],
        out_specs=pl.BlockSpec((BN, HF), lambda nb: (nb, 0)),
        out_shape=jax.ShapeDtypeStruct((N, HF), jnp.float32),
    )(z, wsum)


@jax.jit
def _run(feat, edge_index_0, edge_index_1,
         W0, al0, ar0, W1, al1, ar1, sW1, sb1, sW2):
    W = jnp.stack([W0, W1])
    al = jnp.stack([al0, al1])
    ar = jnp.stack([ar0, ar1])
    eye = jnp.eye(H, dtype=jnp.float32)
    ALb = (al[:, :, :, None] * eye[None, :, None, :]).reshape(P, HF, H)
    ARb = (ar[:, :, :, None] * eye[None, :, None, :]).reshape(P, HF, H)
    pad = jnp.zeros((P, HF, 112), jnp.float32)
    ALR = jnp.concatenate([ALb, ARb, pad], axis=-1)

    h4, elr = _stage1(feat, W, ALR)

    src0 = edge_index_0[0]
    dst0 = edge_index_0[1]
    src1 = edge_index_1[0]
    dst1 = edge_index_1[1]

    ee, dpart = _stats(elr[0], elr[1], src0, dst0, src1, dst1)
    src2d = jnp.concatenate([src0, src1]).reshape(2 * E // ECH, ECH)
    dst2d = jnp.concatenate([dst0, dst1]).reshape(2 * E // ECH, ECH)
    raw = _agg(h4.reshape(P * 4 * N, 128), ee.reshape(P * E, 16),
               src2d, dst2d)

    SEL = np.zeros((4, 128, 128), np.float32)
    for hp in range(4):
        SEL[hp, 2 * hp, 0:64] = 1.0
        SEL[hp, 2 * hp + 1, 64:128] = 1.0
    z, wsum = _c1(raw.reshape(P, 4, N, 128), dpart.reshape(P, N, 128),
                  sW1, sb1.reshape(1, 128), sW2, jnp.asarray(SEL))
    return _c2(z, wsum)


def kernel(feat, edge_index_0, edge_index_1, edge_idx,
           W0, al0, ar0, W1, al1, ar1, sW1, sb1, sW2):
    del edge_idx
    return _run(feat, edge_index_0, edge_index_1,
                W0, al0, ar0, W1, al1, ar1, sW1, sb1, sW2)
